# baseline (speedup 1.0000x reference)
import functools

import jax
import jax.numpy as jnp
from jax import lax
from jax.experimental import pallas as pl
from jax.experimental.pallas import tpu as pltpu
from jax.experimental.pallas import tpu_sc as plsc

_STEP = 10
_POS_THRESH = 0.5

_N = 128 * 8192
_NS = 16
_L = 16
_PER_W = _N // _NS
_CHUNKS = _PER_W // _L

_ROUNDS = (
    (21, 1023, 1024, 31),
    (11, 1023, 1024, 21),
    (0, 2047, 2048, 11),
)
_MAXB = 2048


def _bce_body(preds_ref, targets_ref, bits_ref, total_ref, k_ref):
    p = preds_ref[...]
    t = targets_ref[...]
    losses = jnp.maximum(p, 0.0) - p * t + jnp.log1p(jnp.exp(-jnp.abs(p)))
    masked = jnp.where(t < _POS_THRESH, losses, 0.0)
    bits = lax.bitcast_convert_type(masked, jnp.int32)
    bits_ref[...] = bits
    count = jnp.sum((bits > 0).astype(jnp.int32))
    k_ref[0, 0] = (count + (_STEP - 1)) // _STEP
    total_ref[0, 0] = jnp.sum(losses)


def _sc_select_body(bits_hbm, kvec_hbm, out_hbm,
                    data_v, hist_v, coll_v, merged_v, tmp_v,
                    kv_v, outv_v, csum_s,
                    shared_part, shared_sums):
    c = lax.axis_index("c")
    s = lax.axis_index("s")
    lane = lax.iota(jnp.int32, _L)
    zero16 = jnp.zeros((_L,), jnp.int32)

    pltpu.sync_copy(bits_hbm.at[pl.ds(s * _PER_W, _PER_W)], data_v)
    pltpu.sync_copy(kvec_hbm, kv_v)
    k_rem = jnp.max(kv_v[...])

    prefix = jnp.int32(0)
    v_acc = jnp.int32(0)

    for (sh_bin, b_mask, nb, sh_match) in _ROUNDS:
        lane_base = lane * nb
        nch = nb // _L

        @plsc.parallel_loop(0, (_L * nb) // _L, unroll=8)
        def _(i):
            hist_v[pl.ds(i * _L, _L)] = zero16

        pfx = prefix
        ones16 = jnp.ones((_L,), jnp.int32)

        @plsc.parallel_loop(0, _CHUNKS, unroll=16)
        def _(i):
            x = data_v[pl.ds(i * _L, _L)]
            m = lax.shift_right_logical(x, sh_match) == pfx
            bins = lax.shift_right_logical(x, sh_bin) & b_mask
            idx = bins + lane_base
            plsc.addupdate_scatter(hist_v, [idx], ones16, mask=m)

        @plsc.parallel_loop(0, nch, unroll=2)
        def _(i):
            acc = hist_v[pl.ds(i * _L, _L)]
            for j in range(1, _L):
                acc = acc + hist_v[pl.ds(j * nb + i * _L, _L)]
            coll_v[pl.ds(i * _L, _L)] = acc

        pltpu.sync_copy(coll_v.at[pl.ds(0, nb)], shared_part.at[s, pl.ds(0, nb)])
        plsc.subcore_barrier()

        def mz_body(i, _):
            merged_v[pl.ds(i * _L, _L)] = zero16
            return 0
        lax.fori_loop(0, nch, mz_body, 0)
        for j in range(_NS):
            pltpu.sync_copy(shared_part.at[j, pl.ds(0, nb)],
                            tmp_v.at[pl.ds(0, nb)])

            @plsc.parallel_loop(0, nch, unroll=4)
            def _(i):
                merged_v[pl.ds(i * _L, _L)] = (
                    merged_v[pl.ds(i * _L, _L)] + tmp_v[pl.ds(i * _L, _L)])
        plsc.subcore_barrier()

        @plsc.parallel_loop(0, nch, unroll=4)
        def _(i):
            csum_s[i] = jnp.sum(merged_v[pl.ds(i * _L, _L)])

        def walk(i, carry):
            run, cc_star, run_excl, found = carry
            cc = nch - 1 - i
            sval = csum_s[cc]
            new_run = run + sval
            cross = jnp.logical_and(jnp.logical_not(found), new_run >= k_rem)
            cc_star = jnp.where(cross, cc, cc_star)
            run_excl = jnp.where(cross, run, run_excl)
            found = jnp.logical_or(found, cross)
            return (new_run, cc_star, run_excl, found)
        _, cc_star, run_excl, _ = lax.fori_loop(
            0, nch, walk,
            (jnp.int32(0), jnp.int32(0), jnp.int32(0), jnp.bool_(False)))

        v = merged_v[pl.ds(cc_star * _L, _L)]
        rsuf = lax.rev(plsc.cumsum(lax.rev(v, (0,))), (0,))
        cond = (rsuf + run_excl) >= k_rem
        b = cc_star * _L + jnp.sum(cond.astype(jnp.int32)) - 1
        cnt_above = run_excl + jnp.sum(jnp.where(cond, 0, v))

        k_rem = k_rem - cnt_above
        prefix = b if sh_match == 31 else (prefix << 10) | b
        v_acc = (v_acc << (10 if sh_bin == 11 else (11 if sh_bin == 0 else 0))) | b

    vbits = v_acc

    @plsc.parallel_loop(0, _CHUNKS, unroll=16,
                        carry=(jnp.zeros((_L,), jnp.float32),
                               jnp.zeros((_L,), jnp.float32)))
    def acc2(i, acc):
        a0, a1 = acc
        x = data_v[pl.ds(i * _L, _L)]
        xf = plsc.bitcast(x, jnp.float32)
        contrib = jnp.where(x >= vbits, xf, jnp.float32(0.0))
        return (a1, a0 + contrib)
    acc = acc2[0] + acc2[1]
    mine = jnp.sum(acc)

    outv_v[...] = jnp.full((_L,), mine, jnp.float32)
    pltpu.sync_copy(outv_v, shared_sums.at[s])
    plsc.subcore_barrier()

    @pl.when(jnp.logical_and(c == 0, s == 0))
    def _():
        total = jnp.zeros((_L,), jnp.float32)
        for j in range(_NS):
            pltpu.sync_copy(shared_sums.at[j], outv_v)
            total = total + outv_v[...]
        outv_v[...] = total
        pltpu.sync_copy(outv_v, out_hbm)


def _sc_select(bits_flat, kvec):
    mesh = plsc.VectorSubcoreMesh(core_axis_name="c", subcore_axis_name="s")
    f = functools.partial(
        pl.kernel,
        out_type=jax.ShapeDtypeStruct((_L,), jnp.float32),
        mesh=mesh,
        compiler_params=pltpu.CompilerParams(
            needs_layout_passes=False, use_tc_tiling_on_sc=False),
        scratch_types=[
            pltpu.VMEM((_PER_W,), jnp.int32),
            pltpu.VMEM((_L * _MAXB,), jnp.int32),
            pltpu.VMEM((_MAXB,), jnp.int32),
            pltpu.VMEM((_MAXB,), jnp.int32),
            pltpu.VMEM((_MAXB,), jnp.int32),
            pltpu.VMEM((_L,), jnp.int32),
            pltpu.VMEM((_L,), jnp.float32),
            pltpu.SMEM((_MAXB // _L,), jnp.int32),
            pltpu.VMEM_SHARED((_NS, _MAXB), jnp.int32),
            pltpu.VMEM_SHARED((_NS, _L), jnp.float32),
        ],
    )(_sc_select_body)
    return f(bits_flat, kvec)


def kernel(preds, targets):
    bits, total, k = pl.pallas_call(
        _bce_body,
        out_shape=(
            jax.ShapeDtypeStruct((128, 8192), jnp.int32),
            jax.ShapeDtypeStruct((1, 1), jnp.float32),
            jax.ShapeDtypeStruct((1, 1), jnp.int32),
        ),
        out_specs=(
            pl.BlockSpec(memory_space=pltpu.VMEM),
            pl.BlockSpec(memory_space=pltpu.SMEM),
            pl.BlockSpec(memory_space=pltpu.SMEM),
        ),
    )(preds, targets)
    kvec = jnp.full((_L,), k[0, 0], jnp.int32)
    dropped = _sc_select(bits.reshape(_N), kvec)
    return (total[0, 0] - dropped[0]) / jnp.float32(_N)


# speedup vs baseline: 1.1671x; 1.1671x over previous
import functools

import jax
import jax.numpy as jnp
from jax import lax
from jax.experimental import pallas as pl
from jax.experimental.pallas import tpu as pltpu
from jax.experimental.pallas import tpu_sc as plsc

_STEP = 10
_POS_THRESH = 0.5

_N = 128 * 8192
_NS = 16
_L = 16
_PER_W = _N // _NS
_CHUNKS = _PER_W // _L

_ROUNDS = (
    (21, 1023, 1024, 31),
    (11, 1023, 1024, 21),
    (0, 2047, 2048, 11),
)
_MAXB = 2048


def _bce_body(preds_ref, targets_ref, bits_ref, total_ref, k_ref):
    p = preds_ref[...]
    t = targets_ref[...]
    losses = jnp.maximum(p, 0.0) - p * t + jnp.log1p(jnp.exp(-jnp.abs(p)))
    masked = jnp.where(t < _POS_THRESH, losses, 0.0)
    bits = lax.bitcast_convert_type(masked, jnp.int32)
    bits_ref[...] = bits
    count = jnp.sum((bits > 0).astype(jnp.int32))
    k_ref[0, 0] = (count + (_STEP - 1)) // _STEP
    total_ref[0, 0] = jnp.sum(losses)


def _sc_select_body(bits_hbm, kvec_hbm, out_hbm,
                    data_v, hist_v, coll_v, merged_v, idx64_v, idx128_v,
                    kv_v, outv_v, csum_s,
                    shared_merged, shared_sums):
    c = lax.axis_index("c")
    s = lax.axis_index("s")
    lane = lax.iota(jnp.int32, _L)
    zero16 = jnp.zeros((_L,), jnp.int32)

    pltpu.sync_copy(bits_hbm.at[pl.ds(s * _PER_W, _PER_W)], data_v)
    pltpu.sync_copy(kvec_hbm, kv_v)
    k_rem = jnp.max(kv_v[...])

    @plsc.parallel_loop(0, _MAXB // _L // _L)
    def _(i):
        idx128_v[pl.ds(i * _L, _L)] = lane + i * _L
    @plsc.parallel_loop(0, 4)
    def _(i):
        idx64_v[pl.ds(i * _L, _L)] = lane + i * _L

    prefix = jnp.int32(0)
    v_acc = jnp.int32(0)

    for (sh_bin, b_mask, nb, sh_match) in _ROUNDS:
        lane_base = lane * nb
        nch = nb // _L

        @plsc.parallel_loop(0, (_L * nb) // _L, unroll=8)
        def _(i):
            hist_v[pl.ds(i * _L, _L)] = zero16

        pfx = prefix
        ones16 = jnp.ones((_L,), jnp.int32)

        @plsc.parallel_loop(0, _CHUNKS, unroll=16)
        def _(i):
            x = data_v[pl.ds(i * _L, _L)]
            m = lax.shift_right_logical(x, sh_match) == pfx
            bins = lax.shift_right_logical(x, sh_bin) & b_mask
            idx = bins + lane_base
            plsc.addupdate_scatter(hist_v, [idx], ones16, mask=m)

        @plsc.parallel_loop(0, nch, unroll=2)
        def _(i):
            acc = hist_v[pl.ds(i * _L, _L)]
            for j in range(1, _L):
                acc = acc + hist_v[pl.ds(j * nb + i * _L, _L)]
            coll_v[i, pl.ds(0, _L)] = acc

        idx_ref = idx64_v if nch == 64 else idx128_v

        @pl.when(s == 0)
        def _():
            pltpu.sync_copy(coll_v.at[pl.ds(0, nch)],
                            shared_merged.at[idx_ref])
        plsc.subcore_barrier()

        @pl.when(s != 0)
        def _():
            pltpu.sync_copy(coll_v.at[pl.ds(0, nch)],
                            shared_merged.at[idx_ref], add=True)
        plsc.subcore_barrier()

        pltpu.sync_copy(shared_merged.at[pl.ds(0, nch)],
                        merged_v.at[pl.ds(0, nch)])

        @plsc.parallel_loop(0, nch, unroll=4)
        def _(i):
            csum_s[i] = jnp.sum(merged_v[i, pl.ds(0, _L)])

        def walk(i, carry):
            run, cc_star, run_excl, found = carry
            cc = nch - 1 - i
            sval = csum_s[cc]
            new_run = run + sval
            cross = jnp.logical_and(jnp.logical_not(found), new_run >= k_rem)
            cc_star = jnp.where(cross, cc, cc_star)
            run_excl = jnp.where(cross, run, run_excl)
            found = jnp.logical_or(found, cross)
            return (new_run, cc_star, run_excl, found)
        _, cc_star, run_excl, _ = lax.fori_loop(
            0, nch, walk,
            (jnp.int32(0), jnp.int32(0), jnp.int32(0), jnp.bool_(False)))

        v = merged_v[cc_star, pl.ds(0, _L)]
        rsuf = lax.rev(plsc.cumsum(lax.rev(v, (0,))), (0,))
        cond = (rsuf + run_excl) >= k_rem
        b = cc_star * _L + jnp.sum(cond.astype(jnp.int32)) - 1
        cnt_above = run_excl + jnp.sum(jnp.where(cond, 0, v))

        k_rem = k_rem - cnt_above
        prefix = b if sh_match == 31 else (prefix << 10) | b
        v_acc = (v_acc << (10 if sh_bin == 11 else (11 if sh_bin == 0 else 0))) | b

    vbits = v_acc

    @plsc.parallel_loop(0, _CHUNKS, unroll=16,
                        carry=(jnp.zeros((_L,), jnp.float32),
                               jnp.zeros((_L,), jnp.float32)))
    def acc2(i, acc):
        a0, a1 = acc
        x = data_v[pl.ds(i * _L, _L)]
        xf = plsc.bitcast(x, jnp.float32)
        contrib = jnp.where(x >= vbits, xf, jnp.float32(0.0))
        return (a1, a0 + contrib)
    acc = acc2[0] + acc2[1]
    mine = jnp.sum(acc)

    outv_v[...] = jnp.full((_L,), mine, jnp.float32)
    pltpu.sync_copy(outv_v, shared_sums.at[s])
    plsc.subcore_barrier()

    @pl.when(jnp.logical_and(c == 0, s == 0))
    def _():
        total = jnp.zeros((_L,), jnp.float32)
        for j in range(_NS):
            pltpu.sync_copy(shared_sums.at[j], outv_v)
            total = total + outv_v[...]
        outv_v[...] = total
        pltpu.sync_copy(outv_v, out_hbm)


def _sc_select(bits_flat, kvec):
    mesh = plsc.VectorSubcoreMesh(core_axis_name="c", subcore_axis_name="s")
    f = functools.partial(
        pl.kernel,
        out_type=jax.ShapeDtypeStruct((_L,), jnp.float32),
        mesh=mesh,
        compiler_params=pltpu.CompilerParams(
            needs_layout_passes=False, use_tc_tiling_on_sc=False),
        scratch_types=[
            pltpu.VMEM((_PER_W,), jnp.int32),
            pltpu.VMEM((_L * _MAXB,), jnp.int32),
            pltpu.VMEM((_MAXB // _L, _L), jnp.int32),
            pltpu.VMEM((_MAXB // _L, _L), jnp.int32),
            pltpu.VMEM((64,), jnp.int32),
            pltpu.VMEM((_MAXB // _L,), jnp.int32),
            pltpu.VMEM((_L,), jnp.int32),
            pltpu.VMEM((_L,), jnp.float32),
            pltpu.SMEM((_MAXB // _L,), jnp.int32),
            pltpu.VMEM_SHARED((_MAXB // _L, _L), jnp.int32),
            pltpu.VMEM_SHARED((_NS, _L), jnp.float32),
        ],
    )(_sc_select_body)
    return f(bits_flat, kvec)


def kernel(preds, targets):
    bits, total, k = pl.pallas_call(
        _bce_body,
        out_shape=(
            jax.ShapeDtypeStruct((128, 8192), jnp.int32),
            jax.ShapeDtypeStruct((1, 1), jnp.float32),
            jax.ShapeDtypeStruct((1, 1), jnp.int32),
        ),
        out_specs=(
            pl.BlockSpec(memory_space=pltpu.VMEM),
            pl.BlockSpec(memory_space=pltpu.SMEM),
            pl.BlockSpec(memory_space=pltpu.SMEM),
        ),
    )(preds, targets)
    kvec = jnp.full((_L,), k[0, 0], jnp.int32)
    dropped = _sc_select(bits.reshape(_N), kvec)
    return (total[0, 0] - dropped[0]) / jnp.float32(_N)


# speedup vs baseline: 1.1995x; 1.0278x over previous
import functools

import jax
import jax.numpy as jnp
from jax import lax
from jax.experimental import pallas as pl
from jax.experimental.pallas import tpu as pltpu
from jax.experimental.pallas import tpu_sc as plsc

_STEP = 10
_POS_THRESH = 0.5

_N = 128 * 8192
_NS = 16
_L = 16
_PER_W = _N // _NS
_CHUNKS = _PER_W // _L

_ROUNDS = (
    (21, 1023, 1024, 31),
    (11, 1023, 1024, 21),
    (0, 2047, 2048, 11),
)
_MAXB = 2048


def _bce_body(preds_ref, targets_ref, bits_ref, total_ref, k_ref):
    p = preds_ref[...]
    t = targets_ref[...]
    losses = jnp.maximum(p, 0.0) - p * t + jnp.log1p(jnp.exp(-jnp.abs(p)))
    masked = jnp.where(t < _POS_THRESH, losses, 0.0)
    bits = lax.bitcast_convert_type(masked, jnp.int32)
    bits_ref[...] = bits
    count = jnp.sum((bits > 0).astype(jnp.int32))
    k_ref[0, 0] = (count + (_STEP - 1)) // _STEP
    total_ref[0, 0] = jnp.sum(losses)


def _sc_select_body(bits_hbm, kvec_hbm, out_hbm,
                    data_v, hist_v, coll_v, merged_v, idx64_v, idx128_v,
                    kv_v, outv_v, csum_s,
                    shared_merged, shared_sums):
    c = lax.axis_index("c")
    s = lax.axis_index("s")
    lane = lax.iota(jnp.int32, _L)
    zero16 = jnp.zeros((_L,), jnp.int32)

    pltpu.sync_copy(bits_hbm.at[pl.ds(s * _PER_W, _PER_W)], data_v)
    pltpu.sync_copy(kvec_hbm, kv_v)
    k_rem = jnp.max(kv_v[...])

    @plsc.parallel_loop(0, _MAXB // _L // _L)
    def _(i):
        idx128_v[pl.ds(i * _L, _L)] = lane + i * _L
    @plsc.parallel_loop(0, 4)
    def _(i):
        idx64_v[pl.ds(i * _L, _L)] = lane + i * _L

    prefix = jnp.int32(0)
    v_acc = jnp.int32(0)

    for (sh_bin, b_mask, nb, sh_match) in _ROUNDS:
        lane_base = lane * nb
        nch = nb // _L

        @plsc.parallel_loop(0, (_L * nb) // _L, unroll=8)
        def _(i):
            hist_v[pl.ds(i * _L, _L)] = zero16

        pfx = prefix
        ones16 = jnp.ones((_L,), jnp.int32)

        @plsc.parallel_loop(0, _CHUNKS, unroll=16)
        def _(i):
            x = data_v[pl.ds(i * _L, _L)]
            m = lax.shift_right_logical(x, sh_match) == pfx
            bins = lax.shift_right_logical(x, sh_bin) & b_mask
            idx = bins + lane_base
            plsc.addupdate_scatter(hist_v, [idx], ones16, mask=m)

        @plsc.parallel_loop(0, nch, unroll=2)
        def _(i):
            acc = hist_v[pl.ds(i * _L, _L)]
            for j in range(1, _L):
                acc = acc + hist_v[pl.ds(j * nb + i * _L, _L)]
            coll_v[i, pl.ds(0, _L)] = acc

        idx_ref = idx64_v if nch == 64 else idx128_v

        @pl.when(s == 0)
        def _():
            pltpu.sync_copy(coll_v.at[pl.ds(0, nch)],
                            shared_merged.at[idx_ref])
        plsc.subcore_barrier()

        @pl.when(s != 0)
        def _():
            pltpu.sync_copy(coll_v.at[pl.ds(0, nch)],
                            shared_merged.at[idx_ref], add=True)
        plsc.subcore_barrier()

        pltpu.sync_copy(shared_merged.at[pl.ds(0, nch)],
                        merged_v.at[pl.ds(0, nch)])

        @plsc.parallel_loop(0, nch, unroll=4)
        def _(i):
            csum_s[i] = jnp.sum(merged_v[i, pl.ds(0, _L)])

        def walk(i, carry):
            run, cc_star, run_excl, found = carry
            cc = nch - 1 - i
            sval = csum_s[cc]
            new_run = run + sval
            cross = jnp.logical_and(jnp.logical_not(found), new_run >= k_rem)
            cc_star = jnp.where(cross, cc, cc_star)
            run_excl = jnp.where(cross, run, run_excl)
            found = jnp.logical_or(found, cross)
            return (new_run, cc_star, run_excl, found)
        _, cc_star, run_excl, _ = lax.fori_loop(
            0, nch, walk,
            (jnp.int32(0), jnp.int32(0), jnp.int32(0), jnp.bool_(False)))

        v = merged_v[cc_star, pl.ds(0, _L)]
        rsuf = lax.rev(plsc.cumsum(lax.rev(v, (0,))), (0,))
        cond = (rsuf + run_excl) >= k_rem
        b = cc_star * _L + jnp.sum(cond.astype(jnp.int32)) - 1
        cnt_above = run_excl + jnp.sum(jnp.where(cond, 0, v))

        k_rem = k_rem - cnt_above
        prefix = b if sh_match == 31 else (prefix << 10) | b
        v_acc = (v_acc << (10 if sh_bin == 11 else (11 if sh_bin == 0 else 0))) | b

    vbits = v_acc

    @plsc.parallel_loop(0, _CHUNKS, unroll=16,
                        carry=(jnp.zeros((_L,), jnp.float32),
                               jnp.zeros((_L,), jnp.float32)))
    def acc2(i, acc):
        a0, a1 = acc
        x = data_v[pl.ds(i * _L, _L)]
        xf = plsc.bitcast(x, jnp.float32)
        contrib = jnp.where(x >= vbits, xf, jnp.float32(0.0))
        return (a1, a0 + contrib)
    acc = acc2[0] + acc2[1]
    mine = jnp.sum(acc)

    outv_v[...] = jnp.full((_L,), mine, jnp.float32)
    pltpu.sync_copy(outv_v, shared_sums.at[s])
    plsc.subcore_barrier()

    @pl.when(jnp.logical_and(c == 0, s == 0))
    def _():
        total = jnp.zeros((_L,), jnp.float32)
        for j in range(_NS):
            pltpu.sync_copy(shared_sums.at[j], outv_v)
            total = total + outv_v[...]
        outv_v[...] = total
        pltpu.sync_copy(outv_v, out_hbm)


def _sc_select(bits_flat, kvec):
    mesh = plsc.VectorSubcoreMesh(core_axis_name="c", subcore_axis_name="s", num_cores=1)
    f = functools.partial(
        pl.kernel,
        out_type=jax.ShapeDtypeStruct((_L,), jnp.float32),
        mesh=mesh,
        compiler_params=pltpu.CompilerParams(
            needs_layout_passes=False, use_tc_tiling_on_sc=False),
        scratch_types=[
            pltpu.VMEM((_PER_W,), jnp.int32),
            pltpu.VMEM((_L * _MAXB,), jnp.int32),
            pltpu.VMEM((_MAXB // _L, _L), jnp.int32),
            pltpu.VMEM((_MAXB // _L, _L), jnp.int32),
            pltpu.VMEM((64,), jnp.int32),
            pltpu.VMEM((_MAXB // _L,), jnp.int32),
            pltpu.VMEM((_L,), jnp.int32),
            pltpu.VMEM((_L,), jnp.float32),
            pltpu.SMEM((_MAXB // _L,), jnp.int32),
            pltpu.VMEM_SHARED((_MAXB // _L, _L), jnp.int32),
            pltpu.VMEM_SHARED((_NS, _L), jnp.float32),
        ],
    )(_sc_select_body)
    return f(bits_flat, kvec)


def kernel(preds, targets):
    bits, total, k = pl.pallas_call(
        _bce_body,
        out_shape=(
            jax.ShapeDtypeStruct((128, 8192), jnp.int32),
            jax.ShapeDtypeStruct((1, 1), jnp.float32),
            jax.ShapeDtypeStruct((1, 1), jnp.int32),
        ),
        out_specs=(
            pl.BlockSpec(memory_space=pltpu.VMEM),
            pl.BlockSpec(memory_space=pltpu.SMEM),
            pl.BlockSpec(memory_space=pltpu.SMEM),
        ),
    )(preds, targets)
    kvec = jnp.full((_L,), k[0, 0], jnp.int32)
    dropped = _sc_select(bits.reshape(_N), kvec)
    return (total[0, 0] - dropped[0]) / jnp.float32(_N)
